# Initial kernel scaffold; baseline (speedup 1.0000x reference)
import jax
import jax.numpy as jnp
from jax.experimental import pallas as pl


def kernel(pix_to_face, bary_coords, attributes):
    raise NotImplementedError("write your pallas kernel here")


# speedup vs baseline: 5.6225x; 5.6225x over previous
import functools

import jax
import jax.numpy as jnp
from jax import lax
from jax.experimental import pallas as pl
from jax.experimental.pallas import tpu as pltpu
from jax.experimental.pallas import tpu_sc as plsc

N, H, W = 4, 512, 512
F, D = 10000, 16
NPIX = N * H * W
PPI = H * W
NC, NS, L = 2, 16, 16
NW = NC * NS
SPAN = NPIX // NW
BLK = 512
NBLK = SPAN // BLK
GCH = 128
NG = BLK // GCH
PAIR = 2 * BLK


def _sc_body(table, idxs, bary, out, idx_v, rows_v, bary_v, acc_v, ones_v,
             isem, gsem, osem):
    wid = lax.axis_index("s") * NC + lax.axis_index("c")
    n_img = wid // 8
    r0 = (wid % 8) * SPAN
    g0 = wid * SPAN

    iota = lax.iota(jnp.int32, L)
    one = jnp.full((L,), 1.0, jnp.float32)
    for k in range(PAIR // L):
        ones_v[pl.ds(k * L, L)] = one

    def stage_copies(b, p):
        row0 = pl.multiple_of(g0 // GCH + b * NG, NG)
        return [
            pltpu.make_async_copy(idxs.at[pl.ds(row0, NG)], idx_v.at[p], isem),
            pltpu.make_async_copy(
                bary.at[pl.ds(pl.multiple_of((g0 + b * BLK) * 3, BLK * 3),
                              BLK * 3)],
                bary_v.at[p], isem),
        ]

    def gather_copies(p):
        return [pltpu.make_async_copy(table.at[idx_v.at[p, j]],
                                      rows_v.at[p, pl.ds(j * GCH, GCH)],
                                      gsem.at[p])
                for j in range(NG)]

    def out_copies(bp, pr):
        cps = []
        for d in range(D):
            cps.append(pltpu.make_async_copy(
                acc_v.at[pl.ds(pl.multiple_of(pr * (D * PAIR) + d * PAIR, PAIR),
                               PAIR)],
                out.at[pl.ds(pl.multiple_of(
                    (n_img * 17 + d) * PPI + r0 + bp * PAIR, PAIR), PAIR)],
                osem.at[pr]))
        cps.append(pltpu.make_async_copy(
            ones_v,
            out.at[pl.ds(pl.multiple_of(
                (n_img * 17 + D) * PPI + r0 + bp * PAIR, PAIR), PAIR)],
            osem.at[pr]))
        return cps

    def compute(p, pr):
        off = pr * (D * PAIR) + p * BLK

        @plsc.parallel_loop(0, BLK // L, 1, unroll=2)
        def group(g):
            base = g * L
            pix = iota + base
            p3 = pix * 3
            w0 = plsc.load_gather(bary_v.at[p], [p3])
            w1 = plsc.load_gather(bary_v.at[p], [p3 + 1])
            w2 = plsc.load_gather(bary_v.at[p], [p3 + 2])
            for d in range(D):
                a0 = plsc.load_gather(rows_v.at[p], [pix, jnp.full((L,), d, jnp.int32)])
                a1 = plsc.load_gather(rows_v.at[p], [pix, jnp.full((L,), L + d, jnp.int32)])
                a2 = plsc.load_gather(rows_v.at[p], [pix, jnp.full((L,), 2 * L + d, jnp.int32)])
                acc_v[pl.ds(off + d * PAIR + base, L)] = a0 * w0 + a1 * w1 + a2 * w2

    for c in stage_copies(0, 0):
        c.start()
    for c in stage_copies(0, 0):
        c.wait()
    for c in gather_copies(0):
        c.start()
    for c in stage_copies(1, 1):
        c.start()

    def block(b, carry):
        p = lax.rem(b, 2)
        pr = lax.rem(lax.div(b, 2), 2)
        @pl.when(b + 1 < NBLK)
        def _():
            for c in stage_copies(b + 1, 1 - p):
                c.wait()
            for c in gather_copies(1 - p):
                c.start()
        @pl.when(jnp.logical_and(p == 0, b >= 4))
        def _():
            for c in out_copies(lax.div(b, 2) - 2, pr):
                c.wait()
        for c in gather_copies(p):
            c.wait()
        compute(p, pr)
        @pl.when(p == 1)
        def _():
            for c in out_copies(lax.div(b, 2), pr):
                c.start()
        @pl.when(b + 2 < NBLK)
        def _():
            for c in stage_copies(b + 2, p):
                c.start()
        return carry

    lax.fori_loop(0, NBLK, block, 0)
    for c in out_copies(NBLK // 2 - 2, 0):
        c.wait()
    for c in out_copies(NBLK // 2 - 1, 1):
        c.wait()


@functools.partial(jax.jit, static_argnames=())
def _blend(table, idxs, bary):
    mesh = plsc.VectorSubcoreMesh(core_axis_name="c", subcore_axis_name="s")
    kern = functools.partial(
        pl.kernel, mesh=mesh,
        out_type=jax.ShapeDtypeStruct((N * (D + 1) * PPI,), jnp.float32),
        scratch_types=[
            pltpu.VMEM((2, NG, GCH), jnp.int32),
            pltpu.VMEM((2, BLK, 3 * D), jnp.float32),
            pltpu.VMEM((2, BLK * 3), jnp.float32),
            pltpu.VMEM((2 * D * PAIR,), jnp.float32),
            pltpu.VMEM((PAIR,), jnp.float32),
            pltpu.SemaphoreType.DMA,
            pltpu.SemaphoreType.DMA((2,)),
            pltpu.SemaphoreType.DMA((2,)),
        ],
        compiler_params=pltpu.CompilerParams(needs_layout_passes=False,
                                             use_tc_tiling_on_sc=False),
    )(_sc_body)
    return kern(table, idxs, bary)


def kernel(pix_to_face, bary_coords, attributes):
    idx = pix_to_face.reshape(NPIX // GCH, GCH).astype(jnp.int32)
    bary = bary_coords.reshape(NPIX * 3).astype(jnp.float32)
    table = attributes.reshape(N * F, 3 * D).astype(jnp.float32)
    out = _blend(table, idx, bary)
    return out.reshape(N, D + 1, H, W)


# speedup vs baseline: 28.8162x; 5.1251x over previous
import functools

import jax
import jax.numpy as jnp
from jax import lax
from jax.experimental import pallas as pl
from jax.experimental.pallas import tpu as pltpu
from jax.experimental.pallas import tpu_sc as plsc

N, H, W = 4, 512, 512
F, D = 10000, 16
NPIX = N * H * W
PPI = H * W
NC, NS, L = 2, 16, 16
NW = NC * NS
SPAN = NPIX // NW
BLK = 512
NBLK = SPAN // BLK
GCH = 128
NG = BLK // GCH
PAIR = 2 * BLK


def _sc_body(table, idxs, bary, out, idx_v, rows_v, bary_v, acc_v, ones_v,
             isem, gsem, osem):
    wid = lax.axis_index("s") * NC + lax.axis_index("c")
    n_img = wid // 8
    r0 = (wid % 8) * SPAN
    g0 = wid * SPAN

    iota = lax.iota(jnp.int32, L)
    one = jnp.full((L,), 1.0, jnp.float32)
    for k in range(PAIR // L):
        ones_v[pl.ds(k * L, L)] = one

    def stage_copies(b, p):
        row0 = pl.multiple_of(g0 // GCH + b * NG, NG)
        return [
            pltpu.make_async_copy(idxs.at[pl.ds(row0, NG)], idx_v.at[p], isem),
            pltpu.make_async_copy(
                bary.at[pl.ds(pl.multiple_of((g0 + b * BLK) * 3, BLK * 3),
                              BLK * 3)],
                bary_v.at[p], isem),
        ]

    def gather_copies(p):
        return [pltpu.make_async_copy(table.at[idx_v.at[p, j]],
                                      rows_v.at[p, pl.ds(j * GCH, GCH)],
                                      gsem.at[p])
                for j in range(NG)]

    def out_copies(bp, pr):
        cps = []
        for d in range(D):
            cps.append(pltpu.make_async_copy(
                acc_v.at[pl.ds(pl.multiple_of(pr * (D * PAIR) + d * PAIR, PAIR),
                               PAIR)],
                out.at[pl.ds(pl.multiple_of(
                    (n_img * 17 + d) * PPI + r0 + bp * PAIR, PAIR), PAIR)],
                osem.at[pr]))
        cps.append(pltpu.make_async_copy(
            ones_v,
            out.at[pl.ds(pl.multiple_of(
                (n_img * 17 + D) * PPI + r0 + bp * PAIR, PAIR), PAIR)],
            osem.at[pr]))
        return cps

    def compute(p, pr):
        off = pr * (D * PAIR) + p * BLK

        @plsc.parallel_loop(0, BLK // L, 1, unroll=2)
        def group(g):
            base = g * L
            pix = iota + base
            w0 = bary_v[p, pl.ds(base, L)]
            w1 = bary_v[p, pl.ds(BLK + base, L)]
            w2 = bary_v[p, pl.ds(2 * BLK + base, L)]
            for d in range(D):
                a0 = plsc.load_gather(rows_v.at[p], [pix, jnp.full((L,), d, jnp.int32)])
                a1 = plsc.load_gather(rows_v.at[p], [pix, jnp.full((L,), L + d, jnp.int32)])
                a2 = plsc.load_gather(rows_v.at[p], [pix, jnp.full((L,), 2 * L + d, jnp.int32)])
                acc_v[pl.ds(off + d * PAIR + base, L)] = a0 * w0 + a1 * w1 + a2 * w2

    for c in stage_copies(0, 0):
        c.start()
    for c in stage_copies(0, 0):
        c.wait()
    for c in gather_copies(0):
        c.start()
    for c in stage_copies(1, 1):
        c.start()

    def block(b, carry):
        p = lax.rem(b, 2)
        pr = lax.rem(lax.div(b, 2), 2)
        @pl.when(b + 1 < NBLK)
        def _():
            for c in stage_copies(b + 1, 1 - p):
                c.wait()
            for c in gather_copies(1 - p):
                c.start()
        @pl.when(jnp.logical_and(p == 0, b >= 4))
        def _():
            for c in out_copies(lax.div(b, 2) - 2, pr):
                c.wait()
        for c in gather_copies(p):
            c.wait()
        compute(p, pr)
        @pl.when(p == 1)
        def _():
            for c in out_copies(lax.div(b, 2), pr):
                c.start()
        @pl.when(b + 2 < NBLK)
        def _():
            for c in stage_copies(b + 2, p):
                c.start()
        return carry

    lax.fori_loop(0, NBLK, block, 0)
    for c in out_copies(NBLK // 2 - 2, 0):
        c.wait()
    for c in out_copies(NBLK // 2 - 1, 1):
        c.wait()


@functools.partial(jax.jit, static_argnames=())
def _blend(table, idxs, bary):
    mesh = plsc.VectorSubcoreMesh(core_axis_name="c", subcore_axis_name="s")
    kern = functools.partial(
        pl.kernel, mesh=mesh,
        out_type=jax.ShapeDtypeStruct((N * (D + 1) * PPI,), jnp.float32),
        scratch_types=[
            pltpu.VMEM((2, NG, GCH), jnp.int32),
            pltpu.VMEM((2, BLK, 3 * D), jnp.float32),
            pltpu.VMEM((2, BLK * 3), jnp.float32),
            pltpu.VMEM((2 * D * PAIR,), jnp.float32),
            pltpu.VMEM((PAIR,), jnp.float32),
            pltpu.SemaphoreType.DMA,
            pltpu.SemaphoreType.DMA((2,)),
            pltpu.SemaphoreType.DMA((2,)),
        ],
        compiler_params=pltpu.CompilerParams(needs_layout_passes=False,
                                             use_tc_tiling_on_sc=False),
    )(_sc_body)
    return kern(table, idxs, bary)


def kernel(pix_to_face, bary_coords, attributes):
    idx = pix_to_face.reshape(NPIX // GCH, GCH).astype(jnp.int32)
    bary = bary_coords[:, :, :, 0, :].transpose(0, 1, 3, 2).reshape(NPIX * 3)
    bary = bary.astype(jnp.float32)
    table = attributes.reshape(N * F, 3 * D).astype(jnp.float32)
    out = _blend(table, idx, bary)
    return out.reshape(N, D + 1, H, W)
